# baseline (speedup 1.0000x reference)
import functools

import jax
import jax.numpy as jnp
from jax import lax
from jax.experimental import pallas as pl
from jax.experimental.pallas import tpu as pltpu
from jax.experimental.pallas import tpu_sc as plsc

D = 64
SPLIT = 120
NBUF = 4
PF = 2


def _embed_kernel(bpw: int, seq: int, idx_hbm, table_hbm, out_hbm,
                  idx_v, rows_v, gsem, wsem):
    c = lax.axis_index("c")
    s = lax.axis_index("s")
    wid = s * 2 + c
    base = wid * bpw

    pltpu.sync_copy(idx_hbm.at[pl.ds(base, bpw)], idx_v)

    rest = seq - SPLIT

    def gather(b, buf):
        pltpu.async_copy(
            table_hbm.at[idx_v.at[b, pl.ds(0, SPLIT)]],
            rows_v.at[buf, pl.ds(0, SPLIT)], gsem.at[buf])
        pltpu.async_copy(
            table_hbm.at[idx_v.at[b, pl.ds(SPLIT, rest)]],
            rows_v.at[buf, pl.ds(SPLIT, rest)], gsem.at[buf])

    def wait_gather(b, buf):
        pltpu.make_async_copy(
            table_hbm.at[idx_v.at[b, pl.ds(0, SPLIT)]],
            rows_v.at[buf, pl.ds(0, SPLIT)], gsem.at[buf]).wait()
        pltpu.make_async_copy(
            table_hbm.at[idx_v.at[b, pl.ds(SPLIT, rest)]],
            rows_v.at[buf, pl.ds(SPLIT, rest)], gsem.at[buf]).wait()

    def writeback(b, buf):
        pltpu.async_copy(rows_v.at[buf], out_hbm.at[base + b], wsem.at[buf])

    def wait_writeback(b, buf):
        pltpu.make_async_copy(
            rows_v.at[buf], out_hbm.at[base + b], wsem.at[buf]).wait()

    for b in range(PF):
        gather(b, b)

    def group(g, _):
        for u in range(NBUF):
            b = g * NBUF + u
            pb = (u + PF) % NBUF

            @pl.when(b + PF < bpw)
            def _():
                @pl.when(b + PF >= NBUF)
                def _():
                    wait_writeback(b + PF - NBUF, pb)
                gather(b + PF, pb)

            wait_gather(b, u)
            writeback(b, u)
        return 0

    lax.fori_loop(0, bpw // NBUF, group, 0, unroll=False)

    for k in range(NBUF):
        b = bpw - NBUF + k
        wait_writeback(b, b % NBUF)


@jax.jit
def kernel(indices, table):
    batch, seq = indices.shape
    info = plsc.get_sparse_core_info()
    nw = info.num_cores * info.num_subcores
    assert batch % nw == 0
    bpw = batch // nw
    assert bpw % NBUF == 0 and PF <= NBUF

    idx = indices.astype(jnp.int32)
    mesh = plsc.VectorSubcoreMesh(core_axis_name="c", subcore_axis_name="s")
    out = pl.kernel(
        functools.partial(_embed_kernel, bpw, seq),
        mesh=mesh,
        out_type=jax.ShapeDtypeStruct((batch, seq, D), jnp.float32),
        compiler_params=pltpu.CompilerParams(use_tc_tiling_on_sc=False),
        scratch_types=[
            pltpu.VMEM((bpw, seq), jnp.int32),
            pltpu.VMEM((NBUF, seq, D), jnp.float32),
            pltpu.SemaphoreType.DMA((NBUF,)),
            pltpu.SemaphoreType.DMA((NBUF,)),
        ],
    )(idx, table)
    return out
